# Initial kernel scaffold; baseline (speedup 1.0000x reference)
import jax
import jax.numpy as jnp
from jax.experimental import pallas as pl


def kernel(x, table, bias):
    raise NotImplementedError("write your pallas kernel here")


# speedup vs baseline: 1.1354x; 1.1354x over previous
import functools

import numpy as np
import jax
import jax.numpy as jnp
from jax import lax
from jax.experimental import pallas as pl
from jax.experimental.pallas import tpu as pltpu
from jax.experimental.pallas import tpu_sc as plsc

_FIELD_DIMS = [100000] * 26
_F = len(_FIELD_DIMS)
_B = 16384
_NROWS = int(np.sum(_FIELD_DIMS))
_NC, _NS = 2, 16
_NW = _NC * _NS
_RPW = _B // _NW
_EPW = _RPW * _F
_CHUNK = 128
_NCHUNK = _EPW // _CHUNK
_GROUPS = _RPW // 16

_OFFS = np.concatenate(([0], np.cumsum(_FIELD_DIMS)[:-1])).astype(np.int32)
_OFFS_TILED = np.tile(_OFFS, _RPW).reshape(_NCHUNK, _CHUNK)

_mesh = plsc.VectorSubcoreMesh(core_axis_name="c", subcore_axis_name="s")


@functools.partial(
    pl.kernel,
    out_type=jax.ShapeDtypeStruct((_B,), jnp.float32),
    mesh=_mesh,
    compiler_params=pltpu.CompilerParams(needs_layout_passes=False),
    scratch_types=[
        pltpu.VMEM((_NCHUNK, _CHUNK), jnp.int32),
        pltpu.VMEM((_NCHUNK, _CHUNK), jnp.int32),
        pltpu.VMEM((_NCHUNK, _CHUNK), jnp.float32),
        pltpu.VMEM((_RPW,), jnp.float32),
        pltpu.VMEM((16,), jnp.float32),
        pltpu.SemaphoreType.DMA,
    ],
)
def _lr_kernel(x_hbm, offs_hbm, tbl_hbm, bias_hbm, out_hbm,
               idx_v, off_v, val_v, out_v, bias_v, sem):
    wid = lax.axis_index("s") * _NC + lax.axis_index("c")

    pltpu.sync_copy(x_hbm.at[wid], idx_v)
    pltpu.sync_copy(offs_hbm, off_v)
    pltpu.sync_copy(bias_hbm, bias_v)

    def _add_offs(j, carry):
        for k in range(_CHUNK // 16):
            s = pl.ds(k * 16, 16)
            idx_v[j, s] = idx_v[j, s] + off_v[j, s]
        return carry
    lax.fori_loop(0, _NCHUNK, _add_offs, 0)

    def _fire(j, carry):
        pltpu.async_copy(tbl_hbm.at[idx_v.at[j]], val_v.at[j], sem)
        return carry
    lax.fori_loop(0, _NCHUNK, _fire, 0)

    def _drain(j, carry):
        pltpu.make_async_copy(tbl_hbm.at[idx_v.at[j]], val_v.at[j], sem).wait()
        return carry
    lax.fori_loop(0, _NCHUNK, _drain, 0)

    bias16 = bias_v[...]
    lane = lax.iota(jnp.int32, 16)

    def _reduce(g, carry):
        p0 = g * (16 * _F) + lane * _F
        acc = jnp.zeros((16,), jnp.float32)
        for f in range(_F):
            p = p0 + f
            rr = lax.shift_right_logical(p, 7)
            cc = jnp.bitwise_and(p, 127)
            acc = acc + plsc.load_gather(val_v, [rr, cc])
        z = acc + bias16
        out_v[pl.ds(g * 16, 16)] = 1.0 / (1.0 + jnp.exp(-z))
        return carry
    lax.fori_loop(0, _GROUPS, _reduce, 0)

    pltpu.sync_copy(out_v, out_hbm.at[pl.ds(wid * _RPW, _RPW)])


def kernel(x, table, bias):
    x3 = x.reshape(_NW, _NCHUNK, _CHUNK)
    tbl = table.reshape(_NROWS)
    bias16 = jnp.broadcast_to(bias.astype(jnp.float32), (16,))
    offs = jnp.asarray(_OFFS_TILED)
    return _lr_kernel(x3, offs, tbl, bias16)
